# baseline (speedup 1.0000x reference)
import functools

import jax
import jax.numpy as jnp
from jax import lax
from jax.experimental import pallas as pl
from jax.experimental.pallas import tpu as pltpu
from jax.experimental.pallas import tpu_sc as plsc

N = 10000
E = 320000
DIM = 128
NUM_CLASSES = 64

NC = 2
NS = 16
NW = NC * NS
CHUNK = 128
CHUNKS = 80
EDGES_PER_TILE = CHUNK * CHUNKS
E_PAD = NW * EDGES_PER_TILE
NPAD = 10112
TRASH = NPAD - 1
ROWS_PER_SUB = NPAD // NS

_sc_mesh = plsc.VectorSubcoreMesh(core_axis_name="c", subcore_axis_name="s")


@functools.partial(
    pl.kernel,
    mesh=_sc_mesh,
    out_type=jax.ShapeDtypeStruct((NC, NPAD, DIM), jnp.float32),
    scratch_types=(
        [pltpu.VMEM((CHUNKS, 1, CHUNK), jnp.int32)]
        + [pltpu.VMEM((1, CHUNK), jnp.int32)] * 8
        + [pltpu.VMEM((CHUNK, DIM), jnp.float32)] * 2
        + [pltpu.VMEM_SHARED((NPAD, DIM), jnp.float32)]
        + [pltpu.SemaphoreType.DMA] * 12
    ),
)
def _sc_agg(x_hbm, src_hbm, dst_hbm, zeros_hbm, out_hbm,
            src_v, d0, d1, d2, d3, d4, d5, d6, d7, rows_a, rows_b, acc,
            sga, sgb, ssa, ssb, si0, si1, si2, si3, si4, si5, si6, si7):
    c = lax.axis_index("c")
    s = lax.axis_index("s")
    tile = c * NS + s
    dring = (d0, d1, d2, d3, d4, d5, d6, d7)
    sring = (si0, si1, si2, si3, si4, si5, si6, si7)
    rows = (rows_a, rows_b)
    gsem = (sga, sgb)
    ssem = (ssa, ssb)
    pltpu.sync_copy(src_hbm.at[tile], src_v)
    for k in range(8):
        pltpu.async_copy(dst_hbm.at[tile, k], dring[k], sring[k])
    pltpu.async_copy(x_hbm.at[src_v.at[0, 0]], rows_a, sga)
    pltpu.async_copy(x_hbm.at[src_v.at[1, 0]], rows_b, sgb)
    pltpu.sync_copy(zeros_hbm.at[pl.ds(s * ROWS_PER_SUB, ROWS_PER_SUB)],
                    acc.at[pl.ds(s * ROWS_PER_SUB, ROWS_PER_SUB)])
    plsc.subcore_barrier()

    def body(i, carry):
        j = 8 * i
        for p in range(4):
            m = j + 2 * p
            ka, kb = 2 * p, 2 * p + 1
            ra, rb = rows[0], rows[1]
            pltpu.make_async_copy(x_hbm.at[src_v.at[m, 0]], ra, sga).wait()
            pltpu.make_async_copy(dst_hbm.at[tile, m], dring[ka],
                                  sring[ka]).wait()
            pltpu.async_copy(ra, acc.at[dring[ka].at[0]], ssa, add=True)
            pltpu.make_async_copy(x_hbm.at[src_v.at[m + 1, 0]], rb, sgb).wait()
            pltpu.make_async_copy(dst_hbm.at[tile, m + 1], dring[kb],
                                  sring[kb]).wait()
            pltpu.async_copy(rb, acc.at[dring[kb].at[0]], ssb, add=True)
            pltpu.make_async_copy(ra, acc.at[dring[ka].at[0]], ssa).wait()

            @pl.when(m + 2 < CHUNKS)
            def _():
                pltpu.async_copy(x_hbm.at[src_v.at[m + 2, 0]], ra, sga)

            @pl.when(m + 8 < CHUNKS)
            def _():
                pltpu.async_copy(dst_hbm.at[tile, m + 8], dring[ka], sring[ka])

            pltpu.make_async_copy(rb, acc.at[dring[kb].at[0]], ssb).wait()

            @pl.when(m + 3 < CHUNKS)
            def _():
                pltpu.async_copy(x_hbm.at[src_v.at[m + 3, 0]], rb, sgb)

            @pl.when(m + 9 < CHUNKS)
            def _():
                pltpu.async_copy(dst_hbm.at[tile, m + 9], dring[kb], sring[kb])

        return carry

    lax.fori_loop(0, CHUNKS // 8, body, 0)
    plsc.subcore_barrier()
    pltpu.sync_copy(acc.at[pl.ds(s * ROWS_PER_SUB, ROWS_PER_SUB)],
                    out_hbm.at[c, pl.ds(s * ROWS_PER_SUB, ROWS_PER_SUB)])


ROW_BLOCK = 1000


def _mlp1_body(x_ref, p_ref, W1_ref, b1_ref, W2_ref, b2_ref, o_ref):
    h = x_ref[...] + p_ref[0] + p_ref[1]
    t = jnp.maximum(
        jnp.dot(h, W1_ref[...], preferred_element_type=jnp.float32) + b1_ref[...],
        0.0)
    y = jnp.dot(t, W2_ref[...], preferred_element_type=jnp.float32) + b2_ref[...]
    o_ref[...] = jnp.maximum(y, 0.0)


def _mlp2_body(x_ref, p_ref, W3_ref, b3_ref, W4_ref, b4_ref, o_ref):
    h = x_ref[...] + p_ref[0] + p_ref[1]
    t = jnp.maximum(
        jnp.dot(h, W3_ref[...], preferred_element_type=jnp.float32) + b3_ref[...],
        0.0)
    z = jnp.dot(t, W4_ref[...], preferred_element_type=jnp.float32) + b4_ref[...]
    z = z - jnp.max(z, axis=-1, keepdims=True)
    ez = jnp.exp(z)
    o_ref[...] = ez / jnp.sum(ez, axis=-1, keepdims=True)


def _mlp1(x, p, W1, b1, W2, b2):
    return pl.pallas_call(
        _mlp1_body,
        grid=(N // ROW_BLOCK,),
        in_specs=[
            pl.BlockSpec((ROW_BLOCK, DIM), lambda i: (i, 0)),
            pl.BlockSpec((NC, ROW_BLOCK, DIM), lambda i: (0, i, 0)),
            pl.BlockSpec((DIM, DIM), lambda i: (0, 0)),
            pl.BlockSpec((1, DIM), lambda i: (0, 0)),
            pl.BlockSpec((DIM, DIM), lambda i: (0, 0)),
            pl.BlockSpec((1, DIM), lambda i: (0, 0)),
        ],
        out_specs=pl.BlockSpec((ROW_BLOCK, DIM), lambda i: (i, 0)),
        out_shape=jax.ShapeDtypeStruct((N, DIM), jnp.float32),
    )(x, p, W1, b1, W2, b2)


def _mlp2(x, p, W3, b3, W4, b4):
    return pl.pallas_call(
        _mlp2_body,
        grid=(N // ROW_BLOCK,),
        in_specs=[
            pl.BlockSpec((ROW_BLOCK, DIM), lambda i: (i, 0)),
            pl.BlockSpec((NC, ROW_BLOCK, DIM), lambda i: (0, i, 0)),
            pl.BlockSpec((DIM, NUM_CLASSES), lambda i: (0, 0)),
            pl.BlockSpec((1, NUM_CLASSES), lambda i: (0, 0)),
            pl.BlockSpec((NUM_CLASSES, NUM_CLASSES), lambda i: (0, 0)),
            pl.BlockSpec((1, NUM_CLASSES), lambda i: (0, 0)),
        ],
        out_specs=pl.BlockSpec((ROW_BLOCK, NUM_CLASSES), lambda i: (i, 0)),
        out_shape=jax.ShapeDtypeStruct((N, NUM_CLASSES), jnp.float32),
    )(x, p, W3, b3, W4, b4)


def kernel(node_embeddings, adjacency_lists, W1, b1, W2, b2, W3, b3, W4, b4):
    x = node_embeddings.astype(jnp.float32)
    adj = adjacency_lists.astype(jnp.int32)
    pad = E_PAD - E
    src3 = jnp.concatenate(
        [adj[0], jnp.zeros((pad,), jnp.int32)]).reshape(NW, CHUNKS, 1, CHUNK)
    trash = N + jnp.arange(pad, dtype=jnp.int32) % (NPAD - N)
    dst3 = jnp.concatenate([adj[1], trash]).reshape(NW, CHUNKS, 1, CHUNK)
    zeros = jnp.zeros((NPAD, DIM), jnp.float32)

    p1 = _sc_agg(x, src3, dst3, zeros)
    x1 = _mlp1(x, p1, W1, b1.reshape(1, DIM), W2, b2.reshape(1, DIM))
    p2 = _sc_agg(x1, src3, dst3, zeros)
    return _mlp2(x1, p2, W3, b3.reshape(1, NUM_CLASSES),
                 W4, b4.reshape(1, NUM_CLASSES))


# speedup vs baseline: 1.0053x; 1.0053x over previous
import functools

import jax
import jax.numpy as jnp
from jax import lax
from jax.experimental import pallas as pl
from jax.experimental.pallas import tpu as pltpu
from jax.experimental.pallas import tpu_sc as plsc

N = 10000
E = 320000
DIM = 128
NUM_CLASSES = 64

NC = 2
NS = 16
NW = NC * NS
CHUNK = 128
CHUNKS = 80
EDGES_PER_TILE = CHUNK * CHUNKS
E_PAD = NW * EDGES_PER_TILE
NPAD = 10112
TRASH = NPAD - 1
ROWS_PER_SUB = NPAD // NS

_sc_mesh = plsc.VectorSubcoreMesh(core_axis_name="c", subcore_axis_name="s")


@functools.partial(
    pl.kernel,
    mesh=_sc_mesh,
    out_type=jax.ShapeDtypeStruct((NC, NPAD, DIM), jnp.float32),
    scratch_types=(
        [pltpu.VMEM((CHUNKS, 1, CHUNK), jnp.int32)]
        + [pltpu.VMEM((1, CHUNK), jnp.int32)] * 8
        + [pltpu.VMEM((CHUNK, DIM), jnp.float32)] * 2
        + [pltpu.VMEM_SHARED((NPAD, DIM), jnp.float32)]
        + [pltpu.SemaphoreType.DMA] * 12
    ),
)
def _sc_agg(x_hbm, src_hbm, dst_hbm, zeros_hbm, out_hbm,
            src_v, d0, d1, d2, d3, d4, d5, d6, d7, rows_a, rows_b, acc,
            sga, sgb, ssa, ssb, si0, si1, si2, si3, si4, si5, si6, si7):
    c = lax.axis_index("c")
    s = lax.axis_index("s")
    tile = c * NS + s
    dring = (d0, d1, d2, d3, d4, d5, d6, d7)
    sring = (si0, si1, si2, si3, si4, si5, si6, si7)
    rows = (rows_a, rows_b)
    gsem = (sga, sgb)
    ssem = (ssa, ssb)
    pltpu.sync_copy(src_hbm.at[tile], src_v)
    for k in range(8):
        pltpu.async_copy(dst_hbm.at[tile, k], dring[k], sring[k])
    pltpu.async_copy(x_hbm.at[src_v.at[0, 0]], rows_a, sga)
    pltpu.async_copy(x_hbm.at[src_v.at[1, 0]], rows_b, sgb)
    pltpu.sync_copy(zeros_hbm.at[pl.ds(s * ROWS_PER_SUB, ROWS_PER_SUB)],
                    acc.at[pl.ds(s * ROWS_PER_SUB, ROWS_PER_SUB)])
    plsc.subcore_barrier()

    def body(i, carry):
        j = 8 * i
        for p in range(4):
            m = j + 2 * p
            ka, kb = 2 * p, 2 * p + 1
            ra, rb = rows[0], rows[1]
            pltpu.make_async_copy(x_hbm.at[src_v.at[m, 0]], ra, sga).wait()
            pltpu.make_async_copy(dst_hbm.at[tile, m], dring[ka],
                                  sring[ka]).wait()
            pltpu.async_copy(ra, acc.at[dring[ka].at[0]], ssa, add=True)
            pltpu.make_async_copy(x_hbm.at[src_v.at[m + 1, 0]], rb, sgb).wait()
            pltpu.make_async_copy(dst_hbm.at[tile, m + 1], dring[kb],
                                  sring[kb]).wait()
            pltpu.make_async_copy(ra, acc.at[dring[ka].at[0]], ssa).wait()
            pltpu.async_copy(rb, acc.at[dring[kb].at[0]], ssb, add=True)

            @pl.when(m + 2 < CHUNKS)
            def _():
                pltpu.async_copy(x_hbm.at[src_v.at[m + 2, 0]], ra, sga)

            @pl.when(m + 8 < CHUNKS)
            def _():
                pltpu.async_copy(dst_hbm.at[tile, m + 8], dring[ka], sring[ka])

            pltpu.make_async_copy(rb, acc.at[dring[kb].at[0]], ssb).wait()

            @pl.when(m + 3 < CHUNKS)
            def _():
                pltpu.async_copy(x_hbm.at[src_v.at[m + 3, 0]], rb, sgb)

            @pl.when(m + 9 < CHUNKS)
            def _():
                pltpu.async_copy(dst_hbm.at[tile, m + 9], dring[kb], sring[kb])

        return carry

    lax.fori_loop(0, CHUNKS // 8, body, 0)
    plsc.subcore_barrier()
    pltpu.sync_copy(acc.at[pl.ds(s * ROWS_PER_SUB, ROWS_PER_SUB)],
                    out_hbm.at[c, pl.ds(s * ROWS_PER_SUB, ROWS_PER_SUB)])


ROW_BLOCK = 1000


def _mlp1_body(x_ref, p_ref, W1_ref, b1_ref, W2_ref, b2_ref, o_ref):
    h = x_ref[...] + p_ref[0] + p_ref[1]
    t = jnp.maximum(
        jnp.dot(h, W1_ref[...], preferred_element_type=jnp.float32) + b1_ref[...],
        0.0)
    y = jnp.dot(t, W2_ref[...], preferred_element_type=jnp.float32) + b2_ref[...]
    o_ref[...] = jnp.maximum(y, 0.0)


def _mlp2_body(x_ref, p_ref, W3_ref, b3_ref, W4_ref, b4_ref, o_ref):
    h = x_ref[...] + p_ref[0] + p_ref[1]
    t = jnp.maximum(
        jnp.dot(h, W3_ref[...], preferred_element_type=jnp.float32) + b3_ref[...],
        0.0)
    z = jnp.dot(t, W4_ref[...], preferred_element_type=jnp.float32) + b4_ref[...]
    z = z - jnp.max(z, axis=-1, keepdims=True)
    ez = jnp.exp(z)
    o_ref[...] = ez / jnp.sum(ez, axis=-1, keepdims=True)


def _mlp1(x, p, W1, b1, W2, b2):
    return pl.pallas_call(
        _mlp1_body,
        grid=(N // ROW_BLOCK,),
        in_specs=[
            pl.BlockSpec((ROW_BLOCK, DIM), lambda i: (i, 0)),
            pl.BlockSpec((NC, ROW_BLOCK, DIM), lambda i: (0, i, 0)),
            pl.BlockSpec((DIM, DIM), lambda i: (0, 0)),
            pl.BlockSpec((1, DIM), lambda i: (0, 0)),
            pl.BlockSpec((DIM, DIM), lambda i: (0, 0)),
            pl.BlockSpec((1, DIM), lambda i: (0, 0)),
        ],
        out_specs=pl.BlockSpec((ROW_BLOCK, DIM), lambda i: (i, 0)),
        out_shape=jax.ShapeDtypeStruct((N, DIM), jnp.float32),
    )(x, p, W1, b1, W2, b2)


def _mlp2(x, p, W3, b3, W4, b4):
    return pl.pallas_call(
        _mlp2_body,
        grid=(N // ROW_BLOCK,),
        in_specs=[
            pl.BlockSpec((ROW_BLOCK, DIM), lambda i: (i, 0)),
            pl.BlockSpec((NC, ROW_BLOCK, DIM), lambda i: (0, i, 0)),
            pl.BlockSpec((DIM, NUM_CLASSES), lambda i: (0, 0)),
            pl.BlockSpec((1, NUM_CLASSES), lambda i: (0, 0)),
            pl.BlockSpec((NUM_CLASSES, NUM_CLASSES), lambda i: (0, 0)),
            pl.BlockSpec((1, NUM_CLASSES), lambda i: (0, 0)),
        ],
        out_specs=pl.BlockSpec((ROW_BLOCK, NUM_CLASSES), lambda i: (i, 0)),
        out_shape=jax.ShapeDtypeStruct((N, NUM_CLASSES), jnp.float32),
    )(x, p, W3, b3, W4, b4)


def kernel(node_embeddings, adjacency_lists, W1, b1, W2, b2, W3, b3, W4, b4):
    x = node_embeddings.astype(jnp.float32)
    adj = adjacency_lists.astype(jnp.int32)
    pad = E_PAD - E
    src3 = jnp.concatenate(
        [adj[0], jnp.zeros((pad,), jnp.int32)]).reshape(NW, CHUNKS, 1, CHUNK)
    trash = N + jnp.arange(pad, dtype=jnp.int32) % (NPAD - N)
    dst3 = jnp.concatenate([adj[1], trash]).reshape(NW, CHUNKS, 1, CHUNK)
    zeros = jnp.zeros((NPAD, DIM), jnp.float32)

    p1 = _sc_agg(x, src3, dst3, zeros)
    x1 = _mlp1(x, p1, W1, b1.reshape(1, DIM), W2, b2.reshape(1, DIM))
    p2 = _sc_agg(x1, src3, dst3, zeros)
    return _mlp2(x1, p2, W3, b3.reshape(1, NUM_CLASSES),
                 W4, b4.reshape(1, NUM_CLASSES))


# speedup vs baseline: 1.0216x; 1.0162x over previous
import functools

import jax
import jax.numpy as jnp
from jax import lax
from jax.experimental import pallas as pl
from jax.experimental.pallas import tpu as pltpu
from jax.experimental.pallas import tpu_sc as plsc

N = 10000
E = 320000
DIM = 128
NUM_CLASSES = 64

NC = 2
NS = 16
NW = NC * NS
CHUNK = 128
CHUNKS = 80
EDGES_PER_TILE = CHUNK * CHUNKS
E_PAD = NW * EDGES_PER_TILE
NPAD = 10112
TRASH = NPAD - 1
ROWS_PER_SUB = NPAD // NS

_sc_mesh = plsc.VectorSubcoreMesh(core_axis_name="c", subcore_axis_name="s")


@functools.partial(
    pl.kernel,
    mesh=_sc_mesh,
    out_type=jax.ShapeDtypeStruct((NC, NPAD, DIM), jnp.float32),
    scratch_types=(
        [pltpu.VMEM((CHUNKS, 1, CHUNK), jnp.int32)]
        + [pltpu.VMEM((1, CHUNK), jnp.int32)] * 8
        + [pltpu.VMEM((CHUNK, DIM), jnp.float32)] * 2
        + [pltpu.VMEM_SHARED((NPAD, DIM), jnp.float32)]
        + [pltpu.SemaphoreType.DMA] * 12
    ),
)
def _sc_agg(x_hbm, src_hbm, dst_hbm, zeros_hbm, out_hbm,
            src_v, d0, d1, d2, d3, d4, d5, d6, d7, rows_a, rows_b, acc,
            sga, sgb, ssa, ssb, si0, si1, si2, si3, si4, si5, si6, si7):
    c = lax.axis_index("c")
    s = lax.axis_index("s")
    tile = c * NS + s
    dring = (d0, d1, d2, d3, d4, d5, d6, d7)
    sring = (si0, si1, si2, si3, si4, si5, si6, si7)
    rows = (rows_a, rows_b)
    gsem = (sga, sgb)
    ssem = (ssa, ssb)
    pltpu.sync_copy(src_hbm.at[tile], src_v)
    for k in range(8):
        pltpu.async_copy(dst_hbm.at[tile, k], dring[k], sring[k])
    pltpu.async_copy(x_hbm.at[src_v.at[0, 0]], rows_a, sga)
    pltpu.async_copy(x_hbm.at[src_v.at[1, 0]], rows_b, sgb)
    pltpu.sync_copy(zeros_hbm.at[pl.ds(s * ROWS_PER_SUB, ROWS_PER_SUB)],
                    acc.at[pl.ds(s * ROWS_PER_SUB, ROWS_PER_SUB)])
    plsc.subcore_barrier()

    def body(i, carry):
        j = 8 * i
        for p in range(4):
            m = j + 2 * p
            ka, kb = 2 * p, 2 * p + 1
            ra, rb = rows[0], rows[1]
            pltpu.make_async_copy(x_hbm.at[src_v.at[m, 0]], ra, sga).wait()
            pltpu.make_async_copy(dst_hbm.at[tile, m], dring[ka],
                                  sring[ka]).wait()
            pltpu.sync_copy(ra, acc.at[dring[ka].at[0]], add=True)

            @pl.when(m + 2 < CHUNKS)
            def _():
                pltpu.async_copy(x_hbm.at[src_v.at[m + 2, 0]], ra, sga)

            @pl.when(m + 8 < CHUNKS)
            def _():
                pltpu.async_copy(dst_hbm.at[tile, m + 8], dring[ka], sring[ka])

            pltpu.make_async_copy(x_hbm.at[src_v.at[m + 1, 0]], rb, sgb).wait()
            pltpu.make_async_copy(dst_hbm.at[tile, m + 1], dring[kb],
                                  sring[kb]).wait()
            pltpu.sync_copy(rb, acc.at[dring[kb].at[0]], add=True)

            @pl.when(m + 3 < CHUNKS)
            def _():
                pltpu.async_copy(x_hbm.at[src_v.at[m + 3, 0]], rb, sgb)

            @pl.when(m + 9 < CHUNKS)
            def _():
                pltpu.async_copy(dst_hbm.at[tile, m + 9], dring[kb], sring[kb])

        return carry

    lax.fori_loop(0, CHUNKS // 8, body, 0)
    plsc.subcore_barrier()
    pltpu.sync_copy(acc.at[pl.ds(s * ROWS_PER_SUB, ROWS_PER_SUB)],
                    out_hbm.at[c, pl.ds(s * ROWS_PER_SUB, ROWS_PER_SUB)])


ROW_BLOCK = 1000


def _mlp1_body(x_ref, p_ref, W1_ref, b1_ref, W2_ref, b2_ref, o_ref):
    h = x_ref[...] + p_ref[0] + p_ref[1]
    t = jnp.maximum(
        jnp.dot(h, W1_ref[...], preferred_element_type=jnp.float32) + b1_ref[...],
        0.0)
    y = jnp.dot(t, W2_ref[...], preferred_element_type=jnp.float32) + b2_ref[...]
    o_ref[...] = jnp.maximum(y, 0.0)


def _mlp2_body(x_ref, p_ref, W3_ref, b3_ref, W4_ref, b4_ref, o_ref):
    h = x_ref[...] + p_ref[0] + p_ref[1]
    t = jnp.maximum(
        jnp.dot(h, W3_ref[...], preferred_element_type=jnp.float32) + b3_ref[...],
        0.0)
    z = jnp.dot(t, W4_ref[...], preferred_element_type=jnp.float32) + b4_ref[...]
    z = z - jnp.max(z, axis=-1, keepdims=True)
    ez = jnp.exp(z)
    o_ref[...] = ez / jnp.sum(ez, axis=-1, keepdims=True)


def _mlp1(x, p, W1, b1, W2, b2):
    return pl.pallas_call(
        _mlp1_body,
        grid=(N // ROW_BLOCK,),
        in_specs=[
            pl.BlockSpec((ROW_BLOCK, DIM), lambda i: (i, 0)),
            pl.BlockSpec((NC, ROW_BLOCK, DIM), lambda i: (0, i, 0)),
            pl.BlockSpec((DIM, DIM), lambda i: (0, 0)),
            pl.BlockSpec((1, DIM), lambda i: (0, 0)),
            pl.BlockSpec((DIM, DIM), lambda i: (0, 0)),
            pl.BlockSpec((1, DIM), lambda i: (0, 0)),
        ],
        out_specs=pl.BlockSpec((ROW_BLOCK, DIM), lambda i: (i, 0)),
        out_shape=jax.ShapeDtypeStruct((N, DIM), jnp.float32),
    )(x, p, W1, b1, W2, b2)


def _mlp2(x, p, W3, b3, W4, b4):
    return pl.pallas_call(
        _mlp2_body,
        grid=(N // ROW_BLOCK,),
        in_specs=[
            pl.BlockSpec((ROW_BLOCK, DIM), lambda i: (i, 0)),
            pl.BlockSpec((NC, ROW_BLOCK, DIM), lambda i: (0, i, 0)),
            pl.BlockSpec((DIM, NUM_CLASSES), lambda i: (0, 0)),
            pl.BlockSpec((1, NUM_CLASSES), lambda i: (0, 0)),
            pl.BlockSpec((NUM_CLASSES, NUM_CLASSES), lambda i: (0, 0)),
            pl.BlockSpec((1, NUM_CLASSES), lambda i: (0, 0)),
        ],
        out_specs=pl.BlockSpec((ROW_BLOCK, NUM_CLASSES), lambda i: (i, 0)),
        out_shape=jax.ShapeDtypeStruct((N, NUM_CLASSES), jnp.float32),
    )(x, p, W3, b3, W4, b4)


def kernel(node_embeddings, adjacency_lists, W1, b1, W2, b2, W3, b3, W4, b4):
    x = node_embeddings.astype(jnp.float32)
    adj = adjacency_lists.astype(jnp.int32)
    pad = E_PAD - E
    src3 = jnp.concatenate(
        [adj[0], jnp.zeros((pad,), jnp.int32)]).reshape(NW, CHUNKS, 1, CHUNK)
    trash = N + jnp.arange(pad, dtype=jnp.int32) % (NPAD - N)
    dst3 = jnp.concatenate([adj[1], trash]).reshape(NW, CHUNKS, 1, CHUNK)
    zeros = jnp.zeros((NPAD, DIM), jnp.float32)

    p1 = _sc_agg(x, src3, dst3, zeros)
    x1 = _mlp1(x, p1, W1, b1.reshape(1, DIM), W2, b2.reshape(1, DIM))
    p2 = _sc_agg(x1, src3, dst3, zeros)
    return _mlp2(x1, p2, W3, b3.reshape(1, NUM_CLASSES),
                 W4, b4.reshape(1, NUM_CLASSES))


# speedup vs baseline: 1.0565x; 1.0341x over previous
import functools

import jax
import jax.numpy as jnp
from jax import lax
from jax.experimental import pallas as pl
from jax.experimental.pallas import tpu as pltpu
from jax.experimental.pallas import tpu_sc as plsc

N = 10000
E = 320000
DIM = 128
NUM_CLASSES = 64

NC = 2
NS = 16
NW = NC * NS
CHUNK = 80
CHUNKS = 128
EDGES_PER_TILE = CHUNK * CHUNKS
E_PAD = NW * EDGES_PER_TILE
NPAD = 10112
TRASH = NPAD - 1
ROWS_PER_SUB = NPAD // NS

_sc_mesh = plsc.VectorSubcoreMesh(core_axis_name="c", subcore_axis_name="s")


@functools.partial(
    pl.kernel,
    mesh=_sc_mesh,
    out_type=jax.ShapeDtypeStruct((NC, NPAD, DIM), jnp.float32),
    scratch_types=(
        [pltpu.VMEM((CHUNKS, 1, CHUNK), jnp.int32)]
        + [pltpu.VMEM((1, CHUNK), jnp.int32)] * 8
        + [pltpu.VMEM((CHUNK, DIM), jnp.float32)] * 2
        + [pltpu.VMEM_SHARED((NPAD, DIM), jnp.float32)]
        + [pltpu.SemaphoreType.DMA] * 12
    ),
)
def _sc_agg(x_hbm, src_hbm, dst_hbm, zeros_hbm, out_hbm,
            src_v, d0, d1, d2, d3, d4, d5, d6, d7, rows_a, rows_b, acc,
            sga, sgb, ssa, ssb, si0, si1, si2, si3, si4, si5, si6, si7):
    c = lax.axis_index("c")
    s = lax.axis_index("s")
    tile = c * NS + s
    dring = (d0, d1, d2, d3, d4, d5, d6, d7)
    sring = (si0, si1, si2, si3, si4, si5, si6, si7)
    rows = (rows_a, rows_b)
    gsem = (sga, sgb)
    ssem = (ssa, ssb)
    pltpu.sync_copy(src_hbm.at[tile], src_v)
    for k in range(8):
        pltpu.async_copy(dst_hbm.at[tile, k], dring[k], sring[k])
    pltpu.async_copy(x_hbm.at[src_v.at[0, 0]], rows_a, sga)
    pltpu.async_copy(x_hbm.at[src_v.at[1, 0]], rows_b, sgb)
    pltpu.sync_copy(zeros_hbm.at[pl.ds(s * ROWS_PER_SUB, ROWS_PER_SUB)],
                    acc.at[pl.ds(s * ROWS_PER_SUB, ROWS_PER_SUB)])
    plsc.subcore_barrier()

    def body(i, carry):
        j = 8 * i
        for p in range(4):
            m = j + 2 * p
            ka, kb = 2 * p, 2 * p + 1
            ra, rb = rows[0], rows[1]
            pltpu.make_async_copy(x_hbm.at[src_v.at[m, 0]], ra, sga).wait()
            pltpu.make_async_copy(dst_hbm.at[tile, m], dring[ka],
                                  sring[ka]).wait()
            pltpu.sync_copy(ra, acc.at[dring[ka].at[0]], add=True)

            @pl.when(m + 2 < CHUNKS)
            def _():
                pltpu.async_copy(x_hbm.at[src_v.at[m + 2, 0]], ra, sga)

            @pl.when(m + 8 < CHUNKS)
            def _():
                pltpu.async_copy(dst_hbm.at[tile, m + 8], dring[ka], sring[ka])

            pltpu.make_async_copy(x_hbm.at[src_v.at[m + 1, 0]], rb, sgb).wait()
            pltpu.make_async_copy(dst_hbm.at[tile, m + 1], dring[kb],
                                  sring[kb]).wait()
            pltpu.sync_copy(rb, acc.at[dring[kb].at[0]], add=True)

            @pl.when(m + 3 < CHUNKS)
            def _():
                pltpu.async_copy(x_hbm.at[src_v.at[m + 3, 0]], rb, sgb)

            @pl.when(m + 9 < CHUNKS)
            def _():
                pltpu.async_copy(dst_hbm.at[tile, m + 9], dring[kb], sring[kb])

        return carry

    lax.fori_loop(0, CHUNKS // 8, body, 0)
    plsc.subcore_barrier()
    pltpu.sync_copy(acc.at[pl.ds(s * ROWS_PER_SUB, ROWS_PER_SUB)],
                    out_hbm.at[c, pl.ds(s * ROWS_PER_SUB, ROWS_PER_SUB)])


ROW_BLOCK = 1000


def _mlp1_body(x_ref, p_ref, W1_ref, b1_ref, W2_ref, b2_ref, o_ref):
    h = x_ref[...] + p_ref[0] + p_ref[1]
    t = jnp.maximum(
        jnp.dot(h, W1_ref[...], preferred_element_type=jnp.float32) + b1_ref[...],
        0.0)
    y = jnp.dot(t, W2_ref[...], preferred_element_type=jnp.float32) + b2_ref[...]
    o_ref[...] = jnp.maximum(y, 0.0)


def _mlp2_body(x_ref, p_ref, W3_ref, b3_ref, W4_ref, b4_ref, o_ref):
    h = x_ref[...] + p_ref[0] + p_ref[1]
    t = jnp.maximum(
        jnp.dot(h, W3_ref[...], preferred_element_type=jnp.float32) + b3_ref[...],
        0.0)
    z = jnp.dot(t, W4_ref[...], preferred_element_type=jnp.float32) + b4_ref[...]
    z = z - jnp.max(z, axis=-1, keepdims=True)
    ez = jnp.exp(z)
    o_ref[...] = ez / jnp.sum(ez, axis=-1, keepdims=True)


def _mlp1(x, p, W1, b1, W2, b2):
    return pl.pallas_call(
        _mlp1_body,
        grid=(N // ROW_BLOCK,),
        in_specs=[
            pl.BlockSpec((ROW_BLOCK, DIM), lambda i: (i, 0)),
            pl.BlockSpec((NC, ROW_BLOCK, DIM), lambda i: (0, i, 0)),
            pl.BlockSpec((DIM, DIM), lambda i: (0, 0)),
            pl.BlockSpec((1, DIM), lambda i: (0, 0)),
            pl.BlockSpec((DIM, DIM), lambda i: (0, 0)),
            pl.BlockSpec((1, DIM), lambda i: (0, 0)),
        ],
        out_specs=pl.BlockSpec((ROW_BLOCK, DIM), lambda i: (i, 0)),
        out_shape=jax.ShapeDtypeStruct((N, DIM), jnp.float32),
    )(x, p, W1, b1, W2, b2)


def _mlp2(x, p, W3, b3, W4, b4):
    return pl.pallas_call(
        _mlp2_body,
        grid=(N // ROW_BLOCK,),
        in_specs=[
            pl.BlockSpec((ROW_BLOCK, DIM), lambda i: (i, 0)),
            pl.BlockSpec((NC, ROW_BLOCK, DIM), lambda i: (0, i, 0)),
            pl.BlockSpec((DIM, NUM_CLASSES), lambda i: (0, 0)),
            pl.BlockSpec((1, NUM_CLASSES), lambda i: (0, 0)),
            pl.BlockSpec((NUM_CLASSES, NUM_CLASSES), lambda i: (0, 0)),
            pl.BlockSpec((1, NUM_CLASSES), lambda i: (0, 0)),
        ],
        out_specs=pl.BlockSpec((ROW_BLOCK, NUM_CLASSES), lambda i: (i, 0)),
        out_shape=jax.ShapeDtypeStruct((N, NUM_CLASSES), jnp.float32),
    )(x, p, W3, b3, W4, b4)


def kernel(node_embeddings, adjacency_lists, W1, b1, W2, b2, W3, b3, W4, b4):
    x = node_embeddings.astype(jnp.float32)
    adj = adjacency_lists.astype(jnp.int32)
    pad = E_PAD - E
    src3 = jnp.concatenate(
        [adj[0], jnp.zeros((pad,), jnp.int32)]).reshape(NW, CHUNKS, 1, CHUNK)
    trash = N + jnp.arange(pad, dtype=jnp.int32) % (NPAD - N)
    dst3 = jnp.concatenate([adj[1], trash]).reshape(NW, CHUNKS, 1, CHUNK)
    zeros = jnp.zeros((NPAD, DIM), jnp.float32)

    p1 = _sc_agg(x, src3, dst3, zeros)
    x1 = _mlp1(x, p1, W1, b1.reshape(1, DIM), W2, b2.reshape(1, DIM))
    p2 = _sc_agg(x1, src3, dst3, zeros)
    return _mlp2(x1, p2, W3, b3.reshape(1, NUM_CLASSES),
                 W4, b4.reshape(1, NUM_CLASSES))


# speedup vs baseline: 2.2334x; 2.1140x over previous
import functools

import jax
import jax.numpy as jnp
from jax import lax
from jax.experimental import pallas as pl
from jax.experimental.pallas import tpu as pltpu
from jax.experimental.pallas import tpu_sc as plsc

N = 10000
E = 320000
DIM = 128
NUM_CLASSES = 64

NC = 2
NS = 16
NW = NC * NS
EDGES_PER_TILE = E // NW
CHUNK = 80
CHUNKS = EDGES_PER_TILE // CHUNK
NPAD = 10240
ROWS_PER_SUB = NPAD // NS

_sc_mesh = plsc.VectorSubcoreMesh(core_axis_name="c", subcore_axis_name="s")


@functools.partial(
    pl.kernel,
    mesh=_sc_mesh,
    out_type=jax.ShapeDtypeStruct((NC, NPAD, DIM), jnp.float32),
    scratch_types=[
        pltpu.VMEM((CHUNKS, CHUNK), jnp.int32),
        pltpu.VMEM((CHUNKS, CHUNK), jnp.int32),
        pltpu.VMEM((CHUNK, DIM), jnp.float32),
        pltpu.VMEM_SHARED((NPAD, DIM), jnp.float32),
        pltpu.SemaphoreType.DMA,
    ],
)
def _sc_agg(x_hbm, src_hbm, dst_hbm, zeros_hbm, out_hbm,
            src_v, dst_v, rows_v, acc, sem):
    c = lax.axis_index("c")
    s = lax.axis_index("s")
    tile = c * NS + s
    pltpu.sync_copy(src_hbm.at[tile], src_v)
    pltpu.sync_copy(dst_hbm.at[tile], dst_v)
    pltpu.sync_copy(zeros_hbm.at[pl.ds(s * ROWS_PER_SUB, ROWS_PER_SUB)],
                    acc.at[pl.ds(s * ROWS_PER_SUB, ROWS_PER_SUB)])
    plsc.subcore_barrier()

    def body(j, carry):
        pltpu.async_copy(x_hbm.at[src_v.at[j]], rows_v, sem).wait()
        pltpu.sync_copy(rows_v, acc.at[dst_v.at[j]], add=True)
        return carry

    lax.fori_loop(0, CHUNKS, body, 0)
    plsc.subcore_barrier()
    pltpu.sync_copy(acc.at[pl.ds(s * ROWS_PER_SUB, ROWS_PER_SUB)],
                    out_hbm.at[c, pl.ds(s * ROWS_PER_SUB, ROWS_PER_SUB)])


ROW_BLOCK = 1000


def _mlp1_body(x_ref, p_ref, W1_ref, b1_ref, W2_ref, b2_ref, o_ref):
    h = x_ref[...] + p_ref[0] + p_ref[1]
    t = jnp.maximum(
        jnp.dot(h, W1_ref[...], preferred_element_type=jnp.float32) + b1_ref[...],
        0.0)
    y = jnp.dot(t, W2_ref[...], preferred_element_type=jnp.float32) + b2_ref[...]
    o_ref[...] = jnp.maximum(y, 0.0)


def _mlp2_body(x_ref, p_ref, W3_ref, b3_ref, W4_ref, b4_ref, o_ref):
    h = x_ref[...] + p_ref[0] + p_ref[1]
    t = jnp.maximum(
        jnp.dot(h, W3_ref[...], preferred_element_type=jnp.float32) + b3_ref[...],
        0.0)
    z = jnp.dot(t, W4_ref[...], preferred_element_type=jnp.float32) + b4_ref[...]
    z = z - jnp.max(z, axis=-1, keepdims=True)
    ez = jnp.exp(z)
    o_ref[...] = ez / jnp.sum(ez, axis=-1, keepdims=True)


def _mlp1(x, p, W1, b1, W2, b2):
    return pl.pallas_call(
        _mlp1_body,
        grid=(N // ROW_BLOCK,),
        in_specs=[
            pl.BlockSpec((ROW_BLOCK, DIM), lambda i: (i, 0)),
            pl.BlockSpec((NC, ROW_BLOCK, DIM), lambda i: (0, i, 0)),
            pl.BlockSpec((DIM, DIM), lambda i: (0, 0)),
            pl.BlockSpec((1, DIM), lambda i: (0, 0)),
            pl.BlockSpec((DIM, DIM), lambda i: (0, 0)),
            pl.BlockSpec((1, DIM), lambda i: (0, 0)),
        ],
        out_specs=pl.BlockSpec((ROW_BLOCK, DIM), lambda i: (i, 0)),
        out_shape=jax.ShapeDtypeStruct((N, DIM), jnp.float32),
    )(x, p, W1, b1, W2, b2)


def _mlp2(x, p, W3, b3, W4, b4):
    return pl.pallas_call(
        _mlp2_body,
        grid=(N // ROW_BLOCK,),
        in_specs=[
            pl.BlockSpec((ROW_BLOCK, DIM), lambda i: (i, 0)),
            pl.BlockSpec((NC, ROW_BLOCK, DIM), lambda i: (0, i, 0)),
            pl.BlockSpec((DIM, NUM_CLASSES), lambda i: (0, 0)),
            pl.BlockSpec((1, NUM_CLASSES), lambda i: (0, 0)),
            pl.BlockSpec((NUM_CLASSES, NUM_CLASSES), lambda i: (0, 0)),
            pl.BlockSpec((1, NUM_CLASSES), lambda i: (0, 0)),
        ],
        out_specs=pl.BlockSpec((ROW_BLOCK, NUM_CLASSES), lambda i: (i, 0)),
        out_shape=jax.ShapeDtypeStruct((N, NUM_CLASSES), jnp.float32),
    )(x, p, W3, b3, W4, b4)


def kernel(node_embeddings, adjacency_lists, W1, b1, W2, b2, W3, b3, W4, b4):
    x = node_embeddings.astype(jnp.float32)
    adj = adjacency_lists.astype(jnp.int32)
    src3 = adj[0].reshape(NW, CHUNKS, CHUNK)
    dst3 = adj[1].reshape(NW, CHUNKS, CHUNK)
    zeros = jnp.zeros((NPAD, DIM), jnp.float32)

    p1 = _sc_agg(x, src3, dst3, zeros)
    x1 = _mlp1(x, p1, W1, b1.reshape(1, DIM), W2, b2.reshape(1, DIM))
    p2 = _sc_agg(x1, src3, dst3, zeros)
    return _mlp2(x1, p2, W3, b3.reshape(1, NUM_CLASSES),
                 W4, b4.reshape(1, NUM_CLASSES))


# speedup vs baseline: 2.5626x; 1.1474x over previous
import functools

import jax
import jax.numpy as jnp
from jax import lax
from jax.experimental import pallas as pl
from jax.experimental.pallas import tpu as pltpu
from jax.experimental.pallas import tpu_sc as plsc

N = 10000
E = 320000
DIM = 128
NUM_CLASSES = 64

NC = 2
NS = 16
NW = NC * NS
EDGES_PER_TILE = E // NW
CHUNK = 125
CHUNKS = EDGES_PER_TILE // CHUNK
NPAD = 10240
ROWS_PER_SUB = NPAD // NS

_sc_mesh = plsc.VectorSubcoreMesh(core_axis_name="c", subcore_axis_name="s")


@functools.partial(
    pl.kernel,
    mesh=_sc_mesh,
    out_type=jax.ShapeDtypeStruct((NC, NPAD, DIM), jnp.float32),
    scratch_types=[
        pltpu.VMEM((CHUNKS, CHUNK), jnp.int32),
        pltpu.VMEM((CHUNKS, CHUNK), jnp.int32),
        pltpu.VMEM((CHUNK, DIM), jnp.float32),
        pltpu.VMEM_SHARED((NPAD, DIM), jnp.float32),
        pltpu.SemaphoreType.DMA,
    ],
)
def _sc_agg(x_hbm, src_hbm, dst_hbm, zeros_hbm, out_hbm,
            src_v, dst_v, rows_v, acc, sem):
    c = lax.axis_index("c")
    s = lax.axis_index("s")
    tile = c * NS + s
    pltpu.sync_copy(src_hbm.at[tile], src_v)
    pltpu.sync_copy(dst_hbm.at[tile], dst_v)
    pltpu.sync_copy(zeros_hbm.at[pl.ds(s * ROWS_PER_SUB, ROWS_PER_SUB)],
                    acc.at[pl.ds(s * ROWS_PER_SUB, ROWS_PER_SUB)])
    plsc.subcore_barrier()

    def body(j, carry):
        pltpu.async_copy(x_hbm.at[src_v.at[j]], rows_v, sem).wait()
        pltpu.sync_copy(rows_v, acc.at[dst_v.at[j]], add=True)
        return carry

    lax.fori_loop(0, CHUNKS, body, 0)
    plsc.subcore_barrier()
    pltpu.sync_copy(acc.at[pl.ds(s * ROWS_PER_SUB, ROWS_PER_SUB)],
                    out_hbm.at[c, pl.ds(s * ROWS_PER_SUB, ROWS_PER_SUB)])


ROW_BLOCK = 1000


def _mlp1_body(x_ref, p_ref, W1_ref, b1_ref, W2_ref, b2_ref, o_ref):
    h = x_ref[...] + p_ref[0] + p_ref[1]
    t = jnp.maximum(
        jnp.dot(h, W1_ref[...], preferred_element_type=jnp.float32) + b1_ref[...],
        0.0)
    y = jnp.dot(t, W2_ref[...], preferred_element_type=jnp.float32) + b2_ref[...]
    o_ref[...] = jnp.maximum(y, 0.0)


def _mlp2_body(x_ref, p_ref, W3_ref, b3_ref, W4_ref, b4_ref, o_ref):
    h = x_ref[...] + p_ref[0] + p_ref[1]
    t = jnp.maximum(
        jnp.dot(h, W3_ref[...], preferred_element_type=jnp.float32) + b3_ref[...],
        0.0)
    z = jnp.dot(t, W4_ref[...], preferred_element_type=jnp.float32) + b4_ref[...]
    z = z - jnp.max(z, axis=-1, keepdims=True)
    ez = jnp.exp(z)
    o_ref[...] = ez / jnp.sum(ez, axis=-1, keepdims=True)


def _mlp1(x, p, W1, b1, W2, b2):
    return pl.pallas_call(
        _mlp1_body,
        grid=(N // ROW_BLOCK,),
        in_specs=[
            pl.BlockSpec((ROW_BLOCK, DIM), lambda i: (i, 0)),
            pl.BlockSpec((NC, ROW_BLOCK, DIM), lambda i: (0, i, 0)),
            pl.BlockSpec((DIM, DIM), lambda i: (0, 0)),
            pl.BlockSpec((1, DIM), lambda i: (0, 0)),
            pl.BlockSpec((DIM, DIM), lambda i: (0, 0)),
            pl.BlockSpec((1, DIM), lambda i: (0, 0)),
        ],
        out_specs=pl.BlockSpec((ROW_BLOCK, DIM), lambda i: (i, 0)),
        out_shape=jax.ShapeDtypeStruct((N, DIM), jnp.float32),
    )(x, p, W1, b1, W2, b2)


def _mlp2(x, p, W3, b3, W4, b4):
    return pl.pallas_call(
        _mlp2_body,
        grid=(N // ROW_BLOCK,),
        in_specs=[
            pl.BlockSpec((ROW_BLOCK, DIM), lambda i: (i, 0)),
            pl.BlockSpec((NC, ROW_BLOCK, DIM), lambda i: (0, i, 0)),
            pl.BlockSpec((DIM, NUM_CLASSES), lambda i: (0, 0)),
            pl.BlockSpec((1, NUM_CLASSES), lambda i: (0, 0)),
            pl.BlockSpec((NUM_CLASSES, NUM_CLASSES), lambda i: (0, 0)),
            pl.BlockSpec((1, NUM_CLASSES), lambda i: (0, 0)),
        ],
        out_specs=pl.BlockSpec((ROW_BLOCK, NUM_CLASSES), lambda i: (i, 0)),
        out_shape=jax.ShapeDtypeStruct((N, NUM_CLASSES), jnp.float32),
    )(x, p, W3, b3, W4, b4)


def kernel(node_embeddings, adjacency_lists, W1, b1, W2, b2, W3, b3, W4, b4):
    x = node_embeddings.astype(jnp.float32)
    adj = adjacency_lists.astype(jnp.int32)
    src3 = adj[0].reshape(NW, CHUNKS, CHUNK)
    dst3 = adj[1].reshape(NW, CHUNKS, CHUNK)
    zeros = jnp.zeros((NPAD, DIM), jnp.float32)

    p1 = _sc_agg(x, src3, dst3, zeros)
    x1 = _mlp1(x, p1, W1, b1.reshape(1, DIM), W2, b2.reshape(1, DIM))
    p2 = _sc_agg(x1, src3, dst3, zeros)
    return _mlp2(x1, p2, W3, b3.reshape(1, NUM_CLASSES),
                 W4, b4.reshape(1, NUM_CLASSES))
